# Initial kernel scaffold; baseline (speedup 1.0000x reference)
import jax
import jax.numpy as jnp
from jax.experimental import pallas as pl


def kernel(x, edge_index, edge_attr, t, w1, b1, ln_w, ln_b, w2, b2):
    raise NotImplementedError("write your pallas kernel here")


# speedup vs baseline: 1.8712x; 1.8712x over previous
import functools

import jax
import jax.numpy as jnp
from jax.experimental import pallas as pl
from jax.experimental.pallas import tpu as pltpu

N_NODES = 10000
D = 128
H = 256
EPS = 1e-7

ROWS_PER_BLOCK = 2000


def _mlp_body(x_ref, num_ref, den_ref, w1_ref, b1_ref, lnw_ref, lnb_ref,
              w2_ref, b2_ref, out_ref):
    x = x_ref[...]
    aggr = num_ref[...] / (den_ref[...] + 1e-16)
    h = x + aggr
    h = jnp.dot(h, w1_ref[...], preferred_element_type=jnp.float32) + b1_ref[...]
    mu = jnp.mean(h, axis=-1, keepdims=True)
    var = jnp.mean((h - mu) ** 2, axis=-1, keepdims=True)
    h = (h - mu) * jax.lax.rsqrt(var + 1e-5) * lnw_ref[...] + lnb_ref[...]
    h = jnp.maximum(h, 0.0)
    out_ref[...] = jnp.dot(h, w2_ref[...], preferred_element_type=jnp.float32) + b2_ref[...]


def _mlp(x, num, den, w1, b1, ln_w, ln_b, w2, b2):
    grid = (N_NODES // ROWS_PER_BLOCK,)
    row_spec = pl.BlockSpec((ROWS_PER_BLOCK, D), lambda i: (i, 0))
    full = lambda shape: pl.BlockSpec(shape, lambda i: tuple(0 for _ in shape))
    return pl.pallas_call(
        _mlp_body,
        grid=grid,
        in_specs=[row_spec, row_spec, row_spec,
                  full((D, H)), full((H,)), full((H,)), full((H,)),
                  full((H, D)), full((D,))],
        out_specs=pl.BlockSpec((ROWS_PER_BLOCK, D), lambda i: (i, 0)),
        out_shape=jax.ShapeDtypeStruct((N_NODES, D), jnp.float32),
    )(x, num, den, w1, b1, ln_w, ln_b, w2, b2)


def kernel(x, edge_index, edge_attr, t, w1, b1, ln_w, ln_b, w2, b2):
    src = edge_index[0]
    dst = edge_index[1]
    m = jnp.maximum(x[src] + edge_attr, 0.0) + EPS
    e = jnp.exp(m * t)
    den = jax.ops.segment_sum(e, dst, num_segments=N_NODES)
    num = jax.ops.segment_sum(e * m, dst, num_segments=N_NODES)
    return _mlp(x, num, den, w1, b1, ln_w, ln_b, w2, b2)


# speedup vs baseline: 2.4690x; 1.3195x over previous
import functools

import jax
import jax.numpy as jnp
from jax import lax
from jax.experimental import pallas as pl
from jax.experimental.pallas import tpu as pltpu
from jax.experimental.pallas import tpu_sc as plsc

N_NODES = 10000
N_EDGES = 320000
D = 128
DH = D // 2
H = 2 * D
EPS = 1e-7

NC = 2
NS = 16
CHUNK = 80
EPT = N_EDGES // NS
NCHUNK = EPT // CHUNK
N_PAD = 10240
RPT = N_PAD // NS

ROWS_PER_BLOCK = 2000


def _make_edge_kernel():
    mesh = plsc.VectorSubcoreMesh(core_axis_name="c", subcore_axis_name="s")

    @functools.partial(
        pl.kernel,
        out_type=jax.ShapeDtypeStruct((NC, N_PAD, 2 * DH), jnp.float32),
        mesh=mesh,
        scratch_types=[
            pltpu.VMEM((CHUNK,), jnp.int32),
            pltpu.VMEM((CHUNK,), jnp.int32),
            pltpu.VMEM((CHUNK, D), jnp.float32),
            pltpu.VMEM((CHUNK, D), jnp.float32),
            pltpu.VMEM((CHUNK, 2 * DH), jnp.float32),
            pltpu.VMEM((16,), jnp.float32),
            pltpu.VMEM_SHARED((N_PAD, 2 * DH), jnp.float32),
            pltpu.SemaphoreType.DMA,
            pltpu.SemaphoreType.DMA,
        ],
    )
    def edge_kernel(x_h, ea_h, src_h, dst_h, t_h, zeros_h, out_h,
                    src_v, dst_v, xg_v, ea_v, o_v, t_v, acc_sh,
                    sem1, sem2):
        c = lax.axis_index("c")
        s = lax.axis_index("s")
        col0 = c * DH

        pltpu.sync_copy(zeros_h.at[pl.ds(s * RPT, RPT)],
                        acc_sh.at[pl.ds(s * RPT, RPT)])
        pltpu.sync_copy(t_h, t_v)
        plsc.subcore_barrier()

        def chunk_body(i, carry):
            base = s * EPT + i * CHUNK
            pltpu.sync_copy(src_h.at[pl.ds(base, CHUNK)], src_v)
            pltpu.sync_copy(dst_h.at[pl.ds(base, CHUNK)], dst_v)
            cp1 = pltpu.async_copy(x_h.at[src_v], xg_v, sem1)
            cp2 = pltpu.async_copy(ea_h.at[pl.ds(base, CHUNK)], ea_v, sem2)
            cp1.wait()
            cp2.wait()
            tv = t_v[...]

            def row_body(j, carry2):
                for k in range(DH // 16):
                    xv = xg_v[j, pl.ds(col0 + k * 16, 16)]
                    ev = ea_v[j, pl.ds(col0 + k * 16, 16)]
                    m = jnp.maximum(xv + ev, 0.0) + EPS
                    e = jnp.exp(m * tv)
                    o_v[j, pl.ds(k * 16, 16)] = e * m
                    o_v[j, pl.ds(DH + k * 16, 16)] = e
                return carry2

            lax.fori_loop(0, CHUNK, row_body, 0)
            pltpu.sync_copy(o_v, acc_sh.at[dst_v], add=True)
            return carry

        lax.fori_loop(0, NCHUNK, chunk_body, 0)
        plsc.subcore_barrier()
        pltpu.sync_copy(acc_sh.at[pl.ds(s * RPT, RPT)],
                        out_h.at[c, pl.ds(s * RPT, RPT)])

    return edge_kernel


_EDGE_KERNEL = _make_edge_kernel()


def _mlp_body(x_ref, sc_ref, w1_ref, b1_ref, lnw_ref, lnb_ref,
              w2_ref, b2_ref, out_ref):
    x = x_ref[...]
    num = jnp.concatenate([sc_ref[0, :, :DH], sc_ref[1, :, :DH]], axis=-1)
    den = jnp.concatenate([sc_ref[0, :, DH:], sc_ref[1, :, DH:]], axis=-1)
    aggr = num / (den + 1e-16)
    h = x + aggr
    h = jnp.dot(h, w1_ref[...], preferred_element_type=jnp.float32) + b1_ref[...]
    mu = jnp.mean(h, axis=-1, keepdims=True)
    var = jnp.mean((h - mu) ** 2, axis=-1, keepdims=True)
    h = (h - mu) * lax.rsqrt(var + 1e-5) * lnw_ref[...] + lnb_ref[...]
    h = jnp.maximum(h, 0.0)
    out_ref[...] = jnp.dot(h, w2_ref[...], preferred_element_type=jnp.float32) + b2_ref[...]


def _mlp(x, sc_out, w1, b1, ln_w, ln_b, w2, b2):
    grid = (N_NODES // ROWS_PER_BLOCK,)
    row_spec = pl.BlockSpec((ROWS_PER_BLOCK, D), lambda i: (i, 0))
    full = lambda shape: pl.BlockSpec(shape, lambda i: tuple(0 for _ in shape))
    return pl.pallas_call(
        _mlp_body,
        grid=grid,
        in_specs=[row_spec,
                  pl.BlockSpec((NC, ROWS_PER_BLOCK, D), lambda i: (0, i, 0)),
                  full((D, H)), full((H,)), full((H,)), full((H,)),
                  full((H, D)), full((D,))],
        out_specs=pl.BlockSpec((ROWS_PER_BLOCK, D), lambda i: (i, 0)),
        out_shape=jax.ShapeDtypeStruct((N_NODES, D), jnp.float32),
    )(x, sc_out, w1, b1, ln_w, ln_b, w2, b2)


def kernel(x, edge_index, edge_attr, t, w1, b1, ln_w, ln_b, w2, b2):
    src = edge_index[0]
    dst = edge_index[1]
    t16 = jnp.full((16,), t, dtype=jnp.float32)
    zeros = jnp.zeros((N_PAD, 2 * DH), dtype=jnp.float32)
    sc_out = _EDGE_KERNEL(x, edge_attr, src, dst, t16, zeros)
    return _mlp(x, sc_out, w1, b1, ln_w, ln_b, w2, b2)


# speedup vs baseline: 3.5956x; 1.4563x over previous
import functools

import jax
import jax.numpy as jnp
from jax import lax
from jax.experimental import pallas as pl
from jax.experimental.pallas import tpu as pltpu
from jax.experimental.pallas import tpu_sc as plsc

N_NODES = 10000
N_EDGES = 320000
D = 128
DH = D // 2
H = 2 * D
EPS = 1e-7

NC = 2
NS = 16
CHUNK = 40
EPT = N_EDGES // NS
NCHUNK = EPT // CHUNK
N_PAD = 10240
RPT = N_PAD // NS

ROWS_PER_BLOCK = 2000


def _make_edge_kernel():
    mesh = plsc.VectorSubcoreMesh(core_axis_name="c", subcore_axis_name="s")

    @functools.partial(
        pl.kernel,
        out_type=jax.ShapeDtypeStruct((NC, N_PAD, 2 * DH), jnp.float32),
        mesh=mesh,
        scratch_types=[
            pltpu.VMEM((4, CHUNK), jnp.int32),
            pltpu.VMEM((4, CHUNK), jnp.int32),
            pltpu.VMEM((2, CHUNK, D), jnp.float32),
            pltpu.VMEM((2, CHUNK, D), jnp.float32),
            pltpu.VMEM((2, CHUNK, 2 * DH), jnp.float32),
            pltpu.VMEM((16,), jnp.float32),
            pltpu.VMEM_SHARED((N_PAD, 2 * DH), jnp.float32),
            [pltpu.SemaphoreType.DMA] * 4,
            [pltpu.SemaphoreType.DMA] * 2,
            [pltpu.SemaphoreType.DMA] * 2,
            [pltpu.SemaphoreType.DMA] * 2,
        ],
    )
    def edge_kernel(x_h, ea_h, src_h, dst_h, t_h, zeros_h, out_h,
                    src_v, dst_v, xg_v, ea_v, o_v, t_v, acc_sh,
                    sem_i, sem_g, sem_e, sem_s):
        c = lax.axis_index("c")
        s = lax.axis_index("s")
        col0 = c * DH
        base0 = s * EPT

        pltpu.sync_copy(zeros_h.at[pl.ds(s * RPT, RPT)],
                        acc_sh.at[pl.ds(s * RPT, RPT)])
        pltpu.sync_copy(t_h, t_v)
        plsc.subcore_barrier()
        tv = t_v[...]

        def idx_descs(ci, q):
            base = base0 + ci * CHUNK
            return (pltpu.make_async_copy(src_h.at[pl.ds(base, CHUNK)],
                                          src_v.at[q], sem_i[q]),
                    pltpu.make_async_copy(dst_h.at[pl.ds(base, CHUNK)],
                                          dst_v.at[q], sem_i[q]))

        def load_descs(ci, b, q):
            return (pltpu.make_async_copy(x_h.at[src_v.at[q]], xg_v.at[b],
                                          sem_g[b]),
                    pltpu.make_async_copy(
                        ea_h.at[pl.ds(base0 + ci * CHUNK, CHUNK)],
                        ea_v.at[b], sem_e[b]))

        def scatter_desc(b, q):
            return pltpu.make_async_copy(o_v.at[b], acc_sh.at[dst_v.at[q]],
                                         sem_s[b])

        def step(ci, qc):
            b, bn, qn, q2 = qc % 2, (qc + 1) % 2, (qc + 1) % 4, (qc + 2) % 4

            @pl.when(ci + 1 < NCHUNK)
            def _():
                for d in idx_descs(ci + 1, qn):
                    d.wait()
                for d in load_descs(ci + 1, bn, qn):
                    d.start()

            for d in load_descs(ci, b, qc):
                d.wait()

            @pl.when(ci >= 2)
            def _():
                scatter_desc(b, (qc + 2) % 4).wait()

            @pl.when(ci + 2 < NCHUNK)
            def _():
                for d in idx_descs(ci + 2, q2):
                    d.start()

            def row_body(j, carry2):
                for k in range(DH // 16):
                    xv = xg_v[b, j, pl.ds(col0 + k * 16, 16)]
                    ev = ea_v[b, j, pl.ds(col0 + k * 16, 16)]
                    m = jnp.maximum(xv + ev, 0.0) + EPS
                    e = jnp.exp(m * tv)
                    o_v[b, j, pl.ds(k * 16, 16)] = e * m
                    o_v[b, j, pl.ds(DH + k * 16, 16)] = e
                return carry2

            lax.fori_loop(0, CHUNK, row_body, 0)
            scatter_desc(b, qc).start(add=True)

        for d in idx_descs(0, 0):
            d.start()
        for d in idx_descs(1, 1):
            d.start()
        for d in idx_descs(0, 0):
            d.wait()
        for d in load_descs(0, 0, 0):
            d.start()

        def outer_body(i, carry):
            for u in range(4):
                step(4 * i + u, u)
            return carry

        lax.fori_loop(0, NCHUNK // 4, outer_body, 0)
        scatter_desc(0, (NCHUNK - 2) % 4).wait()
        scatter_desc(1, (NCHUNK - 1) % 4).wait()
        plsc.subcore_barrier()
        pltpu.sync_copy(acc_sh.at[pl.ds(s * RPT, RPT)],
                        out_h.at[c, pl.ds(s * RPT, RPT)])

    return edge_kernel


_EDGE_KERNEL = _make_edge_kernel()


def _mlp_body(x_ref, sc_ref, w1_ref, b1_ref, lnw_ref, lnb_ref,
              w2_ref, b2_ref, out_ref):
    x = x_ref[...]
    num = jnp.concatenate([sc_ref[0, :, :DH], sc_ref[1, :, :DH]], axis=-1)
    den = jnp.concatenate([sc_ref[0, :, DH:], sc_ref[1, :, DH:]], axis=-1)
    aggr = num / (den + 1e-16)
    h = x + aggr
    h = jnp.dot(h, w1_ref[...], preferred_element_type=jnp.float32) + b1_ref[...]
    mu = jnp.mean(h, axis=-1, keepdims=True)
    var = jnp.mean((h - mu) ** 2, axis=-1, keepdims=True)
    h = (h - mu) * lax.rsqrt(var + 1e-5) * lnw_ref[...] + lnb_ref[...]
    h = jnp.maximum(h, 0.0)
    out_ref[...] = jnp.dot(h, w2_ref[...], preferred_element_type=jnp.float32) + b2_ref[...]


def _mlp(x, sc_out, w1, b1, ln_w, ln_b, w2, b2):
    grid = (N_NODES // ROWS_PER_BLOCK,)
    row_spec = pl.BlockSpec((ROWS_PER_BLOCK, D), lambda i: (i, 0))
    full = lambda shape: pl.BlockSpec(shape, lambda i: tuple(0 for _ in shape))
    return pl.pallas_call(
        _mlp_body,
        grid=grid,
        in_specs=[row_spec,
                  pl.BlockSpec((NC, ROWS_PER_BLOCK, D), lambda i: (0, i, 0)),
                  full((D, H)), full((H,)), full((H,)), full((H,)),
                  full((H, D)), full((D,))],
        out_specs=pl.BlockSpec((ROWS_PER_BLOCK, D), lambda i: (i, 0)),
        out_shape=jax.ShapeDtypeStruct((N_NODES, D), jnp.float32),
    )(x, sc_out, w1, b1, ln_w, ln_b, w2, b2)


def kernel(x, edge_index, edge_attr, t, w1, b1, ln_w, ln_b, w2, b2):
    src = edge_index[0]
    dst = edge_index[1]
    t16 = jnp.full((16,), t, dtype=jnp.float32)
    zeros = jnp.zeros((N_PAD, 2 * DH), dtype=jnp.float32)
    sc_out = _EDGE_KERNEL(x, edge_attr, src, dst, t16, zeros)
    return _mlp(x, sc_out, w1, b1, ln_w, ln_b, w2, b2)


# speedup vs baseline: 10.6607x; 2.9649x over previous
import functools

import jax
import jax.numpy as jnp
from jax import lax
from jax.experimental import pallas as pl
from jax.experimental.pallas import tpu as pltpu
from jax.experimental.pallas import tpu_sc as plsc

N_NODES = 10000
N_EDGES = 320000
D = 128
DH = D // 2
H = 2 * D
EPS = 1e-7

NC = 2
NS = 16
CHUNK = 40
EPT = N_EDGES // NS
NCHUNK = EPT // CHUNK
N_PAD = 10240
RPT = N_PAD // NS

ROWS_PER_BLOCK = 2000


def _make_edge_kernel():
    mesh = plsc.VectorSubcoreMesh(core_axis_name="c", subcore_axis_name="s")

    @functools.partial(
        pl.kernel,
        out_type=jax.ShapeDtypeStruct((NC, N_PAD, 2 * DH), jnp.float32),
        mesh=mesh,
        scratch_types=[
            pltpu.VMEM((4, CHUNK), jnp.int32),
            pltpu.VMEM((4, CHUNK), jnp.int32),
            pltpu.VMEM((2, CHUNK, D), jnp.float32),
            pltpu.VMEM((2, CHUNK, D), jnp.float32),
            pltpu.VMEM((2, CHUNK, 2 * DH), jnp.float32),
            pltpu.VMEM((16,), jnp.float32),
            pltpu.VMEM_SHARED((N_PAD, 2 * DH), jnp.float32),
            [pltpu.SemaphoreType.DMA] * 4,
            [pltpu.SemaphoreType.DMA] * 2,
            [pltpu.SemaphoreType.DMA] * 2,
            [pltpu.SemaphoreType.DMA] * 2,
        ],
    )
    def edge_kernel(x_h, ea_h, src_h, dst_h, t_h, zeros_h, out_h,
                    src_v, dst_v, xg_v, ea_v, o_v, t_v, acc_sh,
                    sem_i, sem_g, sem_e, sem_s):
        c = lax.axis_index("c")
        s = lax.axis_index("s")
        col0 = c * DH
        base0 = s * EPT

        pltpu.sync_copy(zeros_h.at[pl.ds(s * RPT, RPT)],
                        acc_sh.at[pl.ds(s * RPT, RPT)])
        pltpu.sync_copy(t_h, t_v)
        plsc.subcore_barrier()
        tv = t_v[...]

        def idx_descs(ci, q):
            base = base0 + ci * CHUNK
            return (pltpu.make_async_copy(src_h.at[pl.ds(base, CHUNK)],
                                          src_v.at[q], sem_i[q]),
                    pltpu.make_async_copy(dst_h.at[pl.ds(base, CHUNK)],
                                          dst_v.at[q], sem_i[q]))

        def load_descs(ci, b, q):
            return (pltpu.make_async_copy(x_h.at[src_v.at[q]], xg_v.at[b],
                                          sem_g[b]),
                    pltpu.make_async_copy(
                        ea_h.at[pl.ds(base0 + ci * CHUNK, CHUNK)],
                        ea_v.at[b], sem_e[b]))

        def scatter_desc(b, q):
            return pltpu.make_async_copy(o_v.at[b], acc_sh.at[dst_v.at[q]],
                                         sem_s[b])

        def step(ci, qc):
            b, bn, qn, q2 = qc % 2, (qc + 1) % 2, (qc + 1) % 4, (qc + 2) % 4

            @pl.when(ci + 1 < NCHUNK)
            def _():
                for d in idx_descs(ci + 1, qn):
                    d.wait()
                for d in load_descs(ci + 1, bn, qn):
                    d.start()

            for d in load_descs(ci, b, qc):
                d.wait()

            @pl.when(ci >= 2)
            def _():
                scatter_desc(b, (qc + 2) % 4).wait()

            @pl.when(ci + 2 < NCHUNK)
            def _():
                for d in idx_descs(ci + 2, q2):
                    d.start()

            @plsc.parallel_loop(0, CHUNK, 1, unroll=4)
            def row_body(j):
                for k in range(DH // 16):
                    xv = xg_v[b, j, pl.ds(col0 + k * 16, 16)]
                    ev = ea_v[b, j, pl.ds(col0 + k * 16, 16)]
                    m = jnp.maximum(xv + ev, 0.0) + EPS
                    e = jnp.exp(m * tv)
                    o_v[b, j, pl.ds(k * 16, 16)] = e * m
                    o_v[b, j, pl.ds(DH + k * 16, 16)] = e

            scatter_desc(b, qc).start(add=True)

        for d in idx_descs(0, 0):
            d.start()
        for d in idx_descs(1, 1):
            d.start()
        for d in idx_descs(0, 0):
            d.wait()
        for d in load_descs(0, 0, 0):
            d.start()

        def outer_body(i, carry):
            for u in range(4):
                step(4 * i + u, u)
            return carry

        lax.fori_loop(0, NCHUNK // 4, outer_body, 0)
        scatter_desc(0, (NCHUNK - 2) % 4).wait()
        scatter_desc(1, (NCHUNK - 1) % 4).wait()
        plsc.subcore_barrier()
        pltpu.sync_copy(acc_sh.at[pl.ds(s * RPT, RPT)],
                        out_h.at[c, pl.ds(s * RPT, RPT)])

    return edge_kernel


_EDGE_KERNEL = _make_edge_kernel()


def _mlp_body(x_ref, sc_ref, w1_ref, b1_ref, lnw_ref, lnb_ref,
              w2_ref, b2_ref, out_ref):
    x = x_ref[...]
    num = jnp.concatenate([sc_ref[0, :, :DH], sc_ref[1, :, :DH]], axis=-1)
    den = jnp.concatenate([sc_ref[0, :, DH:], sc_ref[1, :, DH:]], axis=-1)
    aggr = num / (den + 1e-16)
    h = x + aggr
    h = jnp.dot(h, w1_ref[...], preferred_element_type=jnp.float32) + b1_ref[...]
    mu = jnp.mean(h, axis=-1, keepdims=True)
    var = jnp.mean((h - mu) ** 2, axis=-1, keepdims=True)
    h = (h - mu) * lax.rsqrt(var + 1e-5) * lnw_ref[...] + lnb_ref[...]
    h = jnp.maximum(h, 0.0)
    out_ref[...] = jnp.dot(h, w2_ref[...], preferred_element_type=jnp.float32) + b2_ref[...]


def _mlp(x, sc_out, w1, b1, ln_w, ln_b, w2, b2):
    grid = (N_NODES // ROWS_PER_BLOCK,)
    row_spec = pl.BlockSpec((ROWS_PER_BLOCK, D), lambda i: (i, 0))
    full = lambda shape: pl.BlockSpec(shape, lambda i: tuple(0 for _ in shape))
    return pl.pallas_call(
        _mlp_body,
        grid=grid,
        in_specs=[row_spec,
                  pl.BlockSpec((NC, ROWS_PER_BLOCK, D), lambda i: (0, i, 0)),
                  full((D, H)), full((H,)), full((H,)), full((H,)),
                  full((H, D)), full((D,))],
        out_specs=pl.BlockSpec((ROWS_PER_BLOCK, D), lambda i: (i, 0)),
        out_shape=jax.ShapeDtypeStruct((N_NODES, D), jnp.float32),
    )(x, sc_out, w1, b1, ln_w, ln_b, w2, b2)


def kernel(x, edge_index, edge_attr, t, w1, b1, ln_w, ln_b, w2, b2):
    src = edge_index[0]
    dst = edge_index[1]
    t16 = jnp.full((16,), t, dtype=jnp.float32)
    zeros = jnp.zeros((N_PAD, 2 * DH), dtype=jnp.float32)
    sc_out = _EDGE_KERNEL(x, edge_attr, src, dst, t16, zeros)
    return _mlp(x, sc_out, w1, b1, ln_w, ln_b, w2, b2)


# speedup vs baseline: 15.8280x; 1.4847x over previous
import functools

import jax
import jax.numpy as jnp
from jax import lax
from jax.experimental import pallas as pl
from jax.experimental.pallas import tpu as pltpu
from jax.experimental.pallas import tpu_sc as plsc

N_NODES = 10000
N_EDGES = 320000
D = 128
DH = D // 2
H = 2 * D
EPS = 1e-7

NC = 2
NS = 16
CHUNK = 80
EPT = N_EDGES // NS
NCHUNK = EPT // CHUNK
N_PAD = 10240
RPT = N_PAD // NS

ROWS_PER_BLOCK = 2000


def _make_edge_kernel():
    mesh = plsc.VectorSubcoreMesh(core_axis_name="c", subcore_axis_name="s")

    @functools.partial(
        pl.kernel,
        out_type=jax.ShapeDtypeStruct((NC, N_PAD, 2 * DH), jnp.float32),
        mesh=mesh,
        compiler_params=pltpu.CompilerParams(use_tc_tiling_on_sc=False),
        scratch_types=[
            pltpu.VMEM((4, CHUNK), jnp.int32),
            pltpu.VMEM((4, CHUNK), jnp.int32),
            pltpu.VMEM((4, CHUNK), jnp.int32),
            pltpu.VMEM((2, CHUNK, DH), jnp.float32),
            pltpu.VMEM((2, CHUNK, DH), jnp.float32),
            pltpu.VMEM((2, CHUNK, 2 * DH), jnp.float32),
            pltpu.VMEM((16,), jnp.float32),
            pltpu.VMEM_SHARED((N_PAD, 2 * DH), jnp.float32),
            [pltpu.SemaphoreType.DMA] * 4,
            [pltpu.SemaphoreType.DMA] * 2,
            [pltpu.SemaphoreType.DMA] * 2,
            [pltpu.SemaphoreType.DMA] * 2,
        ],
    )
    def edge_kernel(x_h, ea_h, src_h, eidx_h, dst_h, t_h, zeros_h, out_h,
                    src_v, eidx_v, dst_v, xg_v, ea_v, o_v, t_v, acc_sh,
                    sem_i, sem_g, sem_e, sem_s):
        c = lax.axis_index("c")
        s = lax.axis_index("s")
        base0 = s * EPT

        pltpu.sync_copy(zeros_h.at[pl.ds(s * RPT, RPT)],
                        acc_sh.at[pl.ds(s * RPT, RPT)])
        pltpu.sync_copy(t_h, t_v)
        plsc.subcore_barrier()
        tv = t_v[...]

        def idx_descs(ci, q):
            base = base0 + ci * CHUNK
            cbase = c * N_EDGES + base
            return (pltpu.make_async_copy(src_h.at[pl.ds(cbase, CHUNK)],
                                          src_v.at[q], sem_i[q]),
                    pltpu.make_async_copy(eidx_h.at[pl.ds(cbase, CHUNK)],
                                          eidx_v.at[q], sem_i[q]),
                    pltpu.make_async_copy(dst_h.at[pl.ds(base, CHUNK)],
                                          dst_v.at[q], sem_i[q]))

        def load_descs(ci, b, q):
            return (pltpu.make_async_copy(x_h.at[src_v.at[q]], xg_v.at[b],
                                          sem_g[b]),
                    pltpu.make_async_copy(ea_h.at[eidx_v.at[q]],
                                          ea_v.at[b], sem_e[b]))

        def scatter_desc(b, q):
            return pltpu.make_async_copy(o_v.at[b], acc_sh.at[dst_v.at[q]],
                                         sem_s[b])

        def step(ci, qc):
            b, bn, qn, q2 = qc % 2, (qc + 1) % 2, (qc + 1) % 4, (qc + 2) % 4

            @pl.when(ci + 1 < NCHUNK)
            def _():
                for d in idx_descs(ci + 1, qn):
                    d.wait()
                for d in load_descs(ci + 1, bn, qn):
                    d.start()

            for d in load_descs(ci, b, qc):
                d.wait()

            @pl.when(ci >= 2)
            def _():
                scatter_desc(b, (qc + 2) % 4).wait()

            @pl.when(ci + 2 < NCHUNK)
            def _():
                for d in idx_descs(ci + 2, q2):
                    d.start()

            @plsc.parallel_loop(0, CHUNK, 1, unroll=4)
            def row_body(j):
                for k in range(DH // 16):
                    xv = xg_v[b, j, pl.ds(k * 16, 16)]
                    ev = ea_v[b, j, pl.ds(k * 16, 16)]
                    m = jnp.maximum(xv + ev, 0.0) + EPS
                    e = jnp.exp(m * tv)
                    o_v[b, j, pl.ds(k * 16, 16)] = e * m
                    o_v[b, j, pl.ds(DH + k * 16, 16)] = e

            scatter_desc(b, qc).start(add=True)

        for d in idx_descs(0, 0):
            d.start()
        for d in idx_descs(1, 1):
            d.start()
        for d in idx_descs(0, 0):
            d.wait()
        for d in load_descs(0, 0, 0):
            d.start()

        def outer_body(i, carry):
            for u in range(4):
                step(4 * i + u, u)
            return carry

        lax.fori_loop(0, NCHUNK // 4, outer_body, 0)
        scatter_desc(0, (NCHUNK - 2) % 4).wait()
        scatter_desc(1, (NCHUNK - 1) % 4).wait()
        plsc.subcore_barrier()
        pltpu.sync_copy(acc_sh.at[pl.ds(s * RPT, RPT)],
                        out_h.at[c, pl.ds(s * RPT, RPT)])

    return edge_kernel


_EDGE_KERNEL = _make_edge_kernel()


def _mlp_body(x_ref, sc_ref, w1_ref, b1_ref, lnw_ref, lnb_ref,
              w2_ref, b2_ref, out_ref):
    x = x_ref[...]
    num = jnp.concatenate([sc_ref[0, :, :DH], sc_ref[1, :, :DH]], axis=-1)
    den = jnp.concatenate([sc_ref[0, :, DH:], sc_ref[1, :, DH:]], axis=-1)
    aggr = num / (den + 1e-16)
    h = x + aggr
    h = jnp.dot(h, w1_ref[...], preferred_element_type=jnp.float32) + b1_ref[...]
    mu = jnp.mean(h, axis=-1, keepdims=True)
    var = jnp.mean((h - mu) ** 2, axis=-1, keepdims=True)
    h = (h - mu) * lax.rsqrt(var + 1e-5) * lnw_ref[...] + lnb_ref[...]
    h = jnp.maximum(h, 0.0)
    out_ref[...] = jnp.dot(h, w2_ref[...], preferred_element_type=jnp.float32) + b2_ref[...]


def _mlp(x, sc_out, w1, b1, ln_w, ln_b, w2, b2):
    grid = (N_NODES // ROWS_PER_BLOCK,)
    row_spec = pl.BlockSpec((ROWS_PER_BLOCK, D), lambda i: (i, 0))
    full = lambda shape: pl.BlockSpec(shape, lambda i: tuple(0 for _ in shape))
    return pl.pallas_call(
        _mlp_body,
        grid=grid,
        in_specs=[row_spec,
                  pl.BlockSpec((NC, ROWS_PER_BLOCK, D), lambda i: (0, i, 0)),
                  full((D, H)), full((H,)), full((H,)), full((H,)),
                  full((H, D)), full((D,))],
        out_specs=pl.BlockSpec((ROWS_PER_BLOCK, D), lambda i: (i, 0)),
        out_shape=jax.ShapeDtypeStruct((N_NODES, D), jnp.float32),
    )(x, sc_out, w1, b1, ln_w, ln_b, w2, b2)


def kernel(x, edge_index, edge_attr, t, w1, b1, ln_w, ln_b, w2, b2):
    src = edge_index[0]
    dst = edge_index[1]
    xr2 = jnp.concatenate([x[:, :DH], x[:, DH:]], axis=0)
    ea2 = edge_attr.reshape(2 * N_EDGES, DH)
    src2 = jnp.concatenate([src, src + N_NODES])
    ids = 2 * jnp.arange(N_EDGES, dtype=jnp.int32)
    eidx = jnp.concatenate([ids, ids + 1])
    t16 = jnp.full((16,), t, dtype=jnp.float32)
    zeros = jnp.zeros((N_PAD, 2 * DH), dtype=jnp.float32)
    sc_out = _EDGE_KERNEL(xr2, ea2, src2, eidx, dst, t16, zeros)
    return _mlp(x, sc_out, w1, b1, ln_w, ln_b, w2, b2)
